# baseline (speedup 1.0000x reference)
import jax
import jax.numpy as jnp
from jax import lax
from jax.experimental import pallas as pl
from jax.experimental.pallas import tpu as pltpu
from jax.experimental.pallas import tpu_sc as plsc

N_NODES = 10000
N_EDGES = 320000
DIM = 128
N_LAYERS = 3

NC = 2
NS = 16
L = 16
HALF = DIM // NC
NP = 10240
TN = NP // NS
NB = 80
EB = 128
G = 8
NG = 20
CHUNKS = G * NG
EPT = CHUNKS * EB
EPAD = NS * EPT
NCH = TN // NB

_F32 = jnp.float32
_I32 = jnp.int32


def _newton_rsqrt(x):
    y = 1.0 / x
    for _ in range(26):
        y = y * (1.5 - 0.5 * x * y * y)
    return y


def _body(src_hbm, dst_hbm, emb_hbm, out_hbm, h_hbm, hs_hbm,
          agg, hs_sp, dgo, dgi, ibs, ibd, gbuf, nbuf, norms, onesv,
          gsem0, gsem1, ssem0, ssem1,
          isems0, isems1, isemd0, isemd1, zsem):
    c = lax.axis_index("c")
    s = lax.axis_index("s")
    nbase = s * TN
    coff = c * NP
    z16 = jnp.zeros((L,), _F32)
    gsem = (gsem0, gsem1)
    ssem = (ssem0, ssem1)
    isems = (isems0, isems1)
    isemd = (isemd0, isemd1)

    def _ones(i, _):
        onesv[pl.ds(i * L, L)] = jnp.ones((L,), _F32)
        return 0
    lax.fori_loop(0, EB // L, _ones, 0)

    def _zero_norm0(i, _):
        norms[0, pl.ds(i * L, L)] = z16
        return 0
    lax.fori_loop(0, TN // L, _zero_norm0, 0)

    def _zero_gbuf1(i, _):
        for k in range(HALF // L):
            gbuf[1, i, pl.ds(k * L, L)] = z16
        return 0

    zslice = gbuf.at[1, pl.ds(0, NB)]

    pltpu.sync_copy(norms.at[0], dgo.at[pl.ds(nbase, TN)])
    pltpu.sync_copy(norms.at[0], dgi.at[pl.ds(nbase, TN)])
    lax.fori_loop(0, NB, _zero_gbuf1, 0)

    def _zero_agg(cb, _):
        pltpu.sync_copy(zslice, agg.at[pl.ds(nbase + cb * NB, NB)])
        return 0
    lax.fori_loop(0, NCH, _zero_agg, 0)
    plsc.subcore_barrier()

    def _load_idx(g, half):
        pltpu.async_copy(src_hbm.at[s, pl.ds(g * G, G)],
                         ibs.at[half], isems[half])
        pltpu.async_copy(dst_hbm.at[s, pl.ds(g * G, G)],
                         ibd.at[half], isemd[half])

    def _wait_idx(half):
        pltpu.make_async_copy(src_hbm.at[s, pl.ds(0, G)],
                              ibs.at[half], isems[half]).wait()
        pltpu.make_async_copy(dst_hbm.at[s, pl.ds(0, G)],
                              ibd.at[half], isemd[half]).wait()

    def _run_groups(process_group):
        _load_idx(0, 0)
        _load_idx(1, 1)

        def _pair(m, _):
            for half in range(2):
                g = 2 * m + half
                _wait_idx(half)
                process_group(half)
                gn = lax.rem(g + 2, NG)
                _load_idx(gn, half)
            return 0
        lax.fori_loop(0, NG // 2, _pair, 0)
        _wait_idx(0)
        _wait_idx(1)

    def _deg_group(half):
        descs = []
        for jj in range(G):
            descs.append(pltpu.async_copy(
                onesv, dgo.at[ibs.at[half, jj]], gsem[0], add=True))
            descs.append(pltpu.async_copy(
                onesv, dgi.at[ibd.at[half, jj]], gsem[1], add=True))
        for d in descs:
            d.wait()

    with jax.named_scope("deg"):
        _run_groups(_deg_group)
        plsc.subcore_barrier()

    pltpu.sync_copy(dgi.at[pl.ds(nbase, TN)], norms.at[0])
    pltpu.sync_copy(dgo.at[pl.ds(nbase, TN)], norms.at[1])

    def _norm(i, _):
        sl = pl.ds(i * L, L)
        x = jnp.maximum(norms[0, sl], 1.0)
        norms[0, sl] = _newton_rsqrt(x)
        x = jnp.maximum(norms[1, sl], 1.0)
        ns = _newton_rsqrt(x)
        norms[1, sl] = ns
        norms[2, sl] = x * ns
        return 0
    lax.fori_loop(0, TN // L, _norm, 0)

    def _rows_scale(layer, cb):
        def _rows(g, _):
            ndv = norms[0, pl.ds(cb * NB + g * L, L)]
            nsv = norms[1, pl.ds(cb * NB + g * L, L)]
            for t in range(L):
                i = g * L + t
                f = nsv[t] if layer == 0 else ndv[t] * nsv[t]
                for k in range(HALF // L):
                    sl = pl.ds(k * L, L)
                    nbuf[0, i, sl] = nbuf[0, i, sl] * f
            return 0
        lax.fori_loop(0, NB // L, _rows, 0)

    def node_phase(layer):
        if layer > 0:
            lax.fori_loop(0, NB, _zero_gbuf1, 0)

        def _chunk(cb, _):
            base = nbase + cb * NB
            if layer == 0:
                pltpu.sync_copy(emb_hbm.at[pl.ds(coff + base, NB)],
                                nbuf.at[0])
            else:
                pltpu.sync_copy(agg.at[pl.ds(base, NB)], nbuf.at[0])
                pltpu.async_copy(zslice, agg.at[pl.ds(base, NB)], zsem)
            _rows_scale(layer, cb)
            pltpu.sync_copy(nbuf.at[0], hs_sp.at[pl.ds(base, NB)])
            if layer > 0:
                pltpu.sync_copy(
                    nbuf.at[0], hs_hbm.at[layer - 1, pl.ds(coff + base, NB)])
            return 0
        lax.fori_loop(0, NCH, _chunk, 0)
        if layer > 0:
            def _drain(cb, _):
                pltpu.make_async_copy(
                    zslice, agg.at[pl.ds(nbase, NB)], zsem).wait()
                return 0
            lax.fori_loop(0, NCH, _drain, 0)

    def final_phase():
        def _chunk(cb, _):
            base = nbase + cb * NB
            pltpu.sync_copy(agg.at[pl.ds(base, NB)], nbuf.at[0])
            pltpu.sync_copy(emb_hbm.at[pl.ds(coff + base, NB)],
                            gbuf.at[0, pl.ds(0, NB)])
            pltpu.sync_copy(hs_hbm.at[0, pl.ds(coff + base, NB)],
                            gbuf.at[1, pl.ds(0, NB)])
            pltpu.sync_copy(hs_hbm.at[1, pl.ds(coff + base, NB)],
                            nbuf.at[1])

            def _rows(g, _):
                ndv = norms[0, pl.ds(cb * NB + g * L, L)]
                sdv = norms[2, pl.ds(cb * NB + g * L, L)]
                for t in range(L):
                    i = g * L + t
                    nd = ndv[t]
                    sd = sdv[t]
                    for k in range(HALF // L):
                        sl = pl.ds(k * L, L)
                        h = nbuf[0, i, sl] * nd
                        o = (gbuf[0, i, sl]
                             + (gbuf[1, i, sl] + nbuf[1, i, sl]) * sd + h)
                        nbuf[0, i, sl] = h
                        gbuf[0, i, sl] = o
                return 0
            lax.fori_loop(0, NB // L, _rows, 0)

            pltpu.sync_copy(gbuf.at[0, pl.ds(0, NB)],
                            out_hbm.at[pl.ds(coff + base, NB)])
            pltpu.sync_copy(nbuf.at[0], h_hbm.at[pl.ds(coff + base, NB)])
            return 0
        lax.fori_loop(0, NCH, _chunk, 0)

    def _edge_group(half):
        dg = {}
        sc = {}

        def _scatter(jj):
            b = jj % 2
            dg[jj].wait()
            sc[jj] = pltpu.async_copy(
                gbuf.at[b], agg.at[ibd.at[half, jj]], ssem[b], add=True)

        for jj in range(G):
            b = jj % 2
            if jj >= 2:
                sc[jj - 2].wait()
            dg[jj] = pltpu.async_copy(
                hs_sp.at[ibs.at[half, jj]], gbuf.at[b], gsem[b])
            if jj >= 1:
                _scatter(jj - 1)
        _scatter(G - 1)
        sc[G - 2].wait()
        sc[G - 1].wait()

    with jax.named_scope("node0"):
        node_phase(0)
        plsc.subcore_barrier()
    for layer in range(1, N_LAYERS + 1):
        with jax.named_scope(f"edge{layer}"):
            _run_groups(_edge_group)
            plsc.subcore_barrier()
        if layer < N_LAYERS:
            with jax.named_scope(f"node{layer}"):
                node_phase(layer)
                plsc.subcore_barrier()
        else:
            with jax.named_scope("final"):
                final_phase()


_lightgcn_sc = pl.kernel(
    _body,
    out_type=(
        jax.ShapeDtypeStruct((NC * NP, HALF), _F32),
        jax.ShapeDtypeStruct((NC * NP, HALF), _F32),
        jax.ShapeDtypeStruct((N_LAYERS - 1, NC * NP, HALF), _F32),
    ),
    mesh=plsc.VectorSubcoreMesh(core_axis_name="c", subcore_axis_name="s"),
    compiler_params=pltpu.CompilerParams(use_tc_tiling_on_sc=False),
    scratch_types=[
        pltpu.VMEM_SHARED((NP, HALF), _F32),
        pltpu.VMEM_SHARED((NP, HALF), _F32),
        pltpu.VMEM_SHARED((NP,), _F32),
        pltpu.VMEM_SHARED((NP,), _F32),
        pltpu.VMEM((2, G, EB), _I32),
        pltpu.VMEM((2, G, EB), _I32),
        pltpu.VMEM((2, EB, HALF), _F32),
        pltpu.VMEM((2, NB, HALF), _F32),
        pltpu.VMEM((3, TN), _F32),
        pltpu.VMEM((EB,), _F32),
    ] + [pltpu.SemaphoreType.DMA] * 9,
)


@jax.jit
def kernel(edge_index, embedding):
    src = edge_index[0].astype(_I32)
    dst = edge_index[1].astype(_I32)
    pad_e = EPAD - N_EDGES
    fill = jnp.full((pad_e,), N_NODES, _I32)
    srcp = jnp.concatenate([src, fill]).reshape(NS, CHUNKS, EB)
    dstp = jnp.concatenate([dst, fill]).reshape(NS, CHUNKS, EB)
    embp = jnp.pad(embedding, ((0, NP - N_NODES), (0, 0)))
    emb_r = embp.reshape(NP, NC, HALF).transpose(1, 0, 2).reshape(NC * NP, HALF)

    out_r, h_r, _ = _lightgcn_sc(srcp, dstp, emb_r)

    def _unsplit(a):
        return (a.reshape(NC, NP, HALF).transpose(1, 0, 2)
                .reshape(NP, DIM)[:N_NODES])

    return (_unsplit(out_r), _unsplit(h_r))


# speedup vs baseline: 1.0312x; 1.0312x over previous
import jax
import jax.numpy as jnp
from jax import lax
from jax.experimental import pallas as pl
from jax.experimental.pallas import tpu as pltpu
from jax.experimental.pallas import tpu_sc as plsc

N_NODES = 10000
N_EDGES = 320000
DIM = 128
N_LAYERS = 3

NC = 2
NS = 16
L = 16
HALF = DIM // NC
NP = 10240
TN = NP // NS
NB = 64
EB = 64
G = 8
NG = 40
CHUNKS = G * NG
EPT = CHUNKS * EB
EPAD = NS * EPT
NCH = TN // NB
RING = 4

_F32 = jnp.float32
_I32 = jnp.int32


def _newton_rsqrt(x):
    y = 1.0 / x
    for _ in range(26):
        y = y * (1.5 - 0.5 * x * y * y)
    return y


def _body(src_hbm, dst_hbm, emb_hbm, out_hbm, h_hbm, hs_hbm,
          agg, hs_sp, dgo, dgi, ibs, ibd, gbuf, nbuf, norms, onesv,
          gsem0, gsem1, gsem2, gsem3, ssem0, ssem1, ssem2, ssem3,
          isems0, isems1, isemd0, isemd1, zsem):
    c = lax.axis_index("c")
    s = lax.axis_index("s")
    nbase = s * TN
    coff = c * NP
    z16 = jnp.zeros((L,), _F32)
    gsem = (gsem0, gsem1, gsem2, gsem3)
    ssem = (ssem0, ssem1, ssem2, ssem3)
    isems = (isems0, isems1)
    isemd = (isemd0, isemd1)

    def _ones(i, _):
        onesv[pl.ds(i * L, L)] = jnp.ones((L,), _F32)
        return 0
    lax.fori_loop(0, EB // L, _ones, 0)

    def _zero_norm0(i, _):
        norms[0, pl.ds(i * L, L)] = z16
        return 0
    lax.fori_loop(0, TN // L, _zero_norm0, 0)

    def _zero_gbuf1(i, _):
        for k in range(HALF // L):
            gbuf[1, i, pl.ds(k * L, L)] = z16
        return 0

    zslice = gbuf.at[1, pl.ds(0, NB)]

    pltpu.sync_copy(norms.at[0], dgo.at[pl.ds(nbase, TN)])
    pltpu.sync_copy(norms.at[0], dgi.at[pl.ds(nbase, TN)])
    lax.fori_loop(0, NB, _zero_gbuf1, 0)

    def _zero_agg(cb, _):
        pltpu.sync_copy(zslice, agg.at[pl.ds(nbase + cb * NB, NB)])
        return 0
    lax.fori_loop(0, NCH, _zero_agg, 0)
    plsc.subcore_barrier()

    def _load_idx(g, half):
        pltpu.async_copy(src_hbm.at[s, pl.ds(g * G, G)],
                         ibs.at[half], isems[half])
        pltpu.async_copy(dst_hbm.at[s, pl.ds(g * G, G)],
                         ibd.at[half], isemd[half])

    def _wait_idx(half):
        pltpu.make_async_copy(src_hbm.at[s, pl.ds(0, G)],
                              ibs.at[half], isems[half]).wait()
        pltpu.make_async_copy(dst_hbm.at[s, pl.ds(0, G)],
                              ibd.at[half], isemd[half]).wait()

    def _run_groups(process_group):
        _load_idx(0, 0)
        _load_idx(1, 1)

        def _pair(m, _):
            for half in range(2):
                g = 2 * m + half
                _wait_idx(half)
                process_group(half)
                gn = lax.rem(g + 2, NG)
                _load_idx(gn, half)
            return 0
        lax.fori_loop(0, NG // 2, _pair, 0)
        _wait_idx(0)
        _wait_idx(1)

    def _deg_group(half):
        descs = []
        for jj in range(G):
            descs.append(pltpu.async_copy(
                onesv, dgo.at[ibs.at[half, jj]], gsem[0], add=True))
            descs.append(pltpu.async_copy(
                onesv, dgi.at[ibd.at[half, jj]], gsem[1], add=True))
        for d in descs:
            d.wait()

    with jax.named_scope("deg"):
        _run_groups(_deg_group)
        plsc.subcore_barrier()

    pltpu.sync_copy(dgi.at[pl.ds(nbase, TN)], norms.at[0])
    pltpu.sync_copy(dgo.at[pl.ds(nbase, TN)], norms.at[1])

    def _norm(i, _):
        sl = pl.ds(i * L, L)
        x = jnp.maximum(norms[0, sl], 1.0)
        norms[0, sl] = _newton_rsqrt(x)
        x = jnp.maximum(norms[1, sl], 1.0)
        ns = _newton_rsqrt(x)
        norms[1, sl] = ns
        norms[2, sl] = x * ns
        return 0
    lax.fori_loop(0, TN // L, _norm, 0)

    def _rows_scale(layer, cb):
        def _rows(g, _):
            ndv = norms[0, pl.ds(cb * NB + g * L, L)]
            nsv = norms[1, pl.ds(cb * NB + g * L, L)]
            for t in range(L):
                i = g * L + t
                f = nsv[t] if layer == 0 else ndv[t] * nsv[t]
                for k in range(HALF // L):
                    sl = pl.ds(k * L, L)
                    nbuf[0, i, sl] = nbuf[0, i, sl] * f
            return 0
        lax.fori_loop(0, NB // L, _rows, 0)

    def node_phase(layer):
        if layer > 0:
            lax.fori_loop(0, NB, _zero_gbuf1, 0)

        def _chunk(cb, _):
            base = nbase + cb * NB
            if layer == 0:
                pltpu.sync_copy(emb_hbm.at[pl.ds(coff + base, NB)],
                                nbuf.at[0])
            else:
                pltpu.sync_copy(agg.at[pl.ds(base, NB)], nbuf.at[0])
                pltpu.async_copy(zslice, agg.at[pl.ds(base, NB)], zsem)
            _rows_scale(layer, cb)
            pltpu.sync_copy(nbuf.at[0], hs_sp.at[pl.ds(base, NB)])
            if layer > 0:
                pltpu.sync_copy(
                    nbuf.at[0], hs_hbm.at[layer - 1, pl.ds(coff + base, NB)])
            return 0
        lax.fori_loop(0, NCH, _chunk, 0)
        if layer > 0:
            def _drain(cb, _):
                pltpu.make_async_copy(
                    zslice, agg.at[pl.ds(nbase, NB)], zsem).wait()
                return 0
            lax.fori_loop(0, NCH, _drain, 0)

    def final_phase():
        def _chunk(cb, _):
            base = nbase + cb * NB
            pltpu.sync_copy(agg.at[pl.ds(base, NB)], nbuf.at[0])
            pltpu.sync_copy(emb_hbm.at[pl.ds(coff + base, NB)],
                            gbuf.at[0, pl.ds(0, NB)])
            pltpu.sync_copy(hs_hbm.at[0, pl.ds(coff + base, NB)],
                            gbuf.at[1, pl.ds(0, NB)])
            pltpu.sync_copy(hs_hbm.at[1, pl.ds(coff + base, NB)],
                            nbuf.at[1])

            def _rows(g, _):
                ndv = norms[0, pl.ds(cb * NB + g * L, L)]
                sdv = norms[2, pl.ds(cb * NB + g * L, L)]
                for t in range(L):
                    i = g * L + t
                    nd = ndv[t]
                    sd = sdv[t]
                    for k in range(HALF // L):
                        sl = pl.ds(k * L, L)
                        h = nbuf[0, i, sl] * nd
                        o = (gbuf[0, i, sl]
                             + (gbuf[1, i, sl] + nbuf[1, i, sl]) * sd + h)
                        nbuf[0, i, sl] = h
                        gbuf[0, i, sl] = o
                return 0
            lax.fori_loop(0, NB // L, _rows, 0)

            pltpu.sync_copy(gbuf.at[0, pl.ds(0, NB)],
                            out_hbm.at[pl.ds(coff + base, NB)])
            pltpu.sync_copy(nbuf.at[0], h_hbm.at[pl.ds(coff + base, NB)])
            return 0
        lax.fori_loop(0, NCH, _chunk, 0)

    def _edge_group(half):
        dg = {}
        sc = {}

        def _scatter(jj):
            b = jj % RING
            dg[jj].wait()
            sc[jj] = pltpu.async_copy(
                gbuf.at[b], agg.at[ibd.at[half, jj]], ssem[b], add=True)

        for jj in range(G):
            b = jj % RING
            if jj >= RING:
                sc[jj - RING].wait()
            dg[jj] = pltpu.async_copy(
                hs_sp.at[ibs.at[half, jj]], gbuf.at[b], gsem[b])
            if jj >= 1:
                _scatter(jj - 1)
        _scatter(G - 1)
        for jj in range(max(0, G - RING), G):
            sc[jj].wait()

    with jax.named_scope("node0"):
        node_phase(0)
        plsc.subcore_barrier()
    for layer in range(1, N_LAYERS + 1):
        with jax.named_scope(f"edge{layer}"):
            _run_groups(_edge_group)
            plsc.subcore_barrier()
        if layer < N_LAYERS:
            with jax.named_scope(f"node{layer}"):
                node_phase(layer)
                plsc.subcore_barrier()
        else:
            with jax.named_scope("final"):
                final_phase()


_lightgcn_sc = pl.kernel(
    _body,
    out_type=(
        jax.ShapeDtypeStruct((NC * NP, HALF), _F32),
        jax.ShapeDtypeStruct((NC * NP, HALF), _F32),
        jax.ShapeDtypeStruct((N_LAYERS - 1, NC * NP, HALF), _F32),
    ),
    mesh=plsc.VectorSubcoreMesh(core_axis_name="c", subcore_axis_name="s"),
    compiler_params=pltpu.CompilerParams(use_tc_tiling_on_sc=False),
    scratch_types=[
        pltpu.VMEM_SHARED((NP, HALF), _F32),
        pltpu.VMEM_SHARED((NP, HALF), _F32),
        pltpu.VMEM_SHARED((NP,), _F32),
        pltpu.VMEM_SHARED((NP,), _F32),
        pltpu.VMEM((2, G, EB), _I32),
        pltpu.VMEM((2, G, EB), _I32),
        pltpu.VMEM((RING, EB, HALF), _F32),
        pltpu.VMEM((2, NB, HALF), _F32),
        pltpu.VMEM((3, TN), _F32),
        pltpu.VMEM((EB,), _F32),
    ] + [pltpu.SemaphoreType.DMA] * 13,
)


@jax.jit
def kernel(edge_index, embedding):
    src = edge_index[0].astype(_I32)
    dst = edge_index[1].astype(_I32)
    pad_e = EPAD - N_EDGES
    fill = jnp.full((pad_e,), N_NODES, _I32)
    srcp = jnp.concatenate([src, fill]).reshape(NS, CHUNKS, EB)
    dstp = jnp.concatenate([dst, fill]).reshape(NS, CHUNKS, EB)
    embp = jnp.pad(embedding, ((0, NP - N_NODES), (0, 0)))
    emb_r = embp.reshape(NP, NC, HALF).transpose(1, 0, 2).reshape(NC * NP, HALF)

    out_r, h_r, _ = _lightgcn_sc(srcp, dstp, emb_r)

    def _unsplit(a):
        return (a.reshape(NC, NP, HALF).transpose(1, 0, 2)
                .reshape(NP, DIM)[:N_NODES])

    return (_unsplit(out_r), _unsplit(h_r))
